# Initial kernel scaffold; baseline (speedup 1.0000x reference)
import functools

import jax
import jax.numpy as jnp
from jax import lax
from jax.experimental import pallas as pl
from jax.experimental.pallas import tpu as pltpu
from jax.experimental.pallas import tpu_sc as plsc

N = 10000
E = 320000
D = 128
NC, NS = 2, 16
NW = NC * NS
CB = 128
TCH = 160
EPAD = NW * TCH * CB
ROWS_PER_TILE = N // NS


def _matmul_body(x_ref, w_ref, o_ref):
    x = x_ref[...]
    o_ref[0] = jnp.dot(x, w_ref[0], preferred_element_type=jnp.float32)
    o_ref[1] = jnp.dot(x, w_ref[1], preferred_element_type=jnp.float32)


def _combine_body(a_ref, b_ref, o_ref):
    o_ref[...] = jnp.maximum(a_ref[0] + a_ref[1] + b_ref[...], 0.0)


def _sc_agg(src_h, dst_h, val_h, pre_h, out_h, src_v, dst_v, val_v, rows_v,
            acc_s, sem):
    cid = lax.axis_index("c")
    sid = lax.axis_index("s")
    wid = sid * NC + cid

    def _zero_rows(e, _):
        zero = jnp.zeros((16,), jnp.float32)
        for d in range(D // 16):
            rows_v[e, pl.ds(d * 16, 16)] = zero
        return 0
    lax.fori_loop(0, CB, _zero_rows, 0)
    base = sid * ROWS_PER_TILE
    nfull = ROWS_PER_TILE // CB
    rem = ROWS_PER_TILE - nfull * CB
    for k in range(nfull):
        pltpu.sync_copy(rows_v, acc_s.at[pl.ds(base + k * CB, CB)])
    pltpu.sync_copy(rows_v.at[pl.ds(0, rem)],
                    acc_s.at[pl.ds(base + nfull * CB, rem)])
    plsc.subcore_barrier()

    pltpu.sync_copy(src_h.at[wid], src_v)
    pltpu.sync_copy(dst_h.at[wid], dst_v)
    pltpu.sync_copy(val_h.at[wid], val_v)

    def _chunk(j, _):
        pltpu.async_copy(pre_h.at[src_v.at[j]], rows_v, sem).wait()

        def _scale(e, _):
            lj = jnp.full((16,), j, jnp.int32)
            le = jnp.full((16,), e, jnp.int32)
            v = plsc.load_gather(val_v, [lj, le])
            for d in range(D // 16):
                sl = (e, pl.ds(d * 16, 16))
                rows_v[sl] = rows_v[sl] * v
            return 0
        lax.fori_loop(0, CB, _scale, 0)

        pltpu.sync_copy(rows_v, acc_s.at[dst_v.at[j]], add=True)
        return 0
    lax.fori_loop(0, TCH, _chunk, 0)

    plsc.subcore_barrier()
    pltpu.sync_copy(acc_s.at[pl.ds(base, ROWS_PER_TILE)],
                    out_h.at[cid, pl.ds(base, ROWS_PER_TILE)])


def kernel(x, edge_index_0, edge_vals_0, edge_index_1, edge_vals_1, W0, W1, b):
    wstk = jnp.stack([W0, W1])
    blk = 1000
    pre = pl.pallas_call(
        _matmul_body,
        grid=(N // blk,),
        in_specs=[
            pl.BlockSpec((blk, D), lambda i: (i, 0)),
            pl.BlockSpec((2, D, D), lambda i: (0, 0, 0)),
        ],
        out_specs=pl.BlockSpec((2, blk, D), lambda i: (0, i, 0)),
        out_shape=jax.ShapeDtypeStruct((2, N, D), jnp.float32),
    )(x, wstk)
    pre_cat = pre.reshape(2 * N, D)

    src = jnp.concatenate([edge_index_0[1], edge_index_1[1] + N])
    dst = jnp.concatenate([edge_index_0[0], edge_index_1[0]])
    val = jnp.concatenate([edge_vals_0, edge_vals_1])
    pad = EPAD - 2 * E
    src = jnp.concatenate([src, jnp.zeros((pad,), jnp.int32)]).reshape(
        NW, TCH, CB)
    dst = jnp.concatenate([dst, jnp.zeros((pad,), jnp.int32)]).reshape(
        NW, TCH, CB)
    val = jnp.concatenate([val, jnp.zeros((pad,), jnp.float32)]).reshape(
        NW, TCH, CB)

    mesh = plsc.VectorSubcoreMesh(core_axis_name="c", subcore_axis_name="s")
    agg = pl.kernel(
        _sc_agg,
        out_type=jax.ShapeDtypeStruct((NC, N, D), jnp.float32),
        mesh=mesh,
        scratch_types=[
            pltpu.VMEM((TCH, CB), jnp.int32),
            pltpu.VMEM((TCH, CB), jnp.int32),
            pltpu.VMEM((TCH, CB), jnp.float32),
            pltpu.VMEM((CB, D), jnp.float32),
            pltpu.VMEM_SHARED((N, D), jnp.float32),
            pltpu.SemaphoreType.DMA,
        ],
    )
    acc = agg(src, dst, val, pre_cat)

    out = pl.pallas_call(
        _combine_body,
        grid=(N // blk,),
        in_specs=[
            pl.BlockSpec((2, blk, D), lambda i: (0, i, 0)),
            pl.BlockSpec((1, D), lambda i: (0, 0)),
        ],
        out_specs=pl.BlockSpec((blk, D), lambda i: (i, 0)),
        out_shape=jax.ShapeDtypeStruct((N, D), jnp.float32),
    )(acc, b.reshape(1, D))
    return out


# speedup vs baseline: 2.2027x; 2.2027x over previous
import jax
import jax.numpy as jnp
from jax import lax
from jax.experimental import pallas as pl
from jax.experimental.pallas import tpu as pltpu
from jax.experimental.pallas import tpu_sc as plsc

N = 10000
E = 320000
D = 128
DH = D // 2
NC, NS = 2, 16
CB = 128
TCH = 320
NBLK, CBLK = 4, 80
EPAD = NS * TCH * CB
ROWS_PER_TILE = 624
TAIL_BASE = NS * ROWS_PER_TILE
TAIL = N - TAIL_BASE


def _matmul_body(x_ref, w_ref, o_ref):
    x = x_ref[...]
    p0 = jnp.dot(x, w_ref[0], preferred_element_type=jnp.float32)
    p1 = jnp.dot(x, w_ref[1], preferred_element_type=jnp.float32)
    o_ref[0, 0] = p0[:, :DH]
    o_ref[0, 1] = p1[:, :DH]
    o_ref[1, 0] = p0[:, DH:]
    o_ref[1, 1] = p1[:, DH:]


def _combine_body(a_ref, b_ref, o_ref):
    full = jnp.concatenate([a_ref[0], a_ref[1]], axis=-1)
    o_ref[...] = jnp.maximum(full + b_ref[...], 0.0)


def _sc_agg(src_h, dst_h, val_h, pre_h, out_h, src_v, dst_v, val_v, rows_v,
            acc_s, sem):
    cid = lax.axis_index("c")
    sid = lax.axis_index("s")

    def _zero_rows(e, _):
        zero = jnp.zeros((16,), jnp.float32)
        for d in range(DH // 16):
            rows_v[e, pl.ds(d * 16, 16)] = zero
        return 0
    lax.fori_loop(0, CB, _zero_rows, 0)
    base = sid * ROWS_PER_TILE
    nfull = ROWS_PER_TILE // CB
    rem = ROWS_PER_TILE - nfull * CB
    for k in range(nfull):
        pltpu.sync_copy(rows_v, acc_s.at[pl.ds(base + k * CB, CB)])
    pltpu.sync_copy(rows_v.at[pl.ds(0, rem)],
                    acc_s.at[pl.ds(base + nfull * CB, rem)])

    @pl.when(sid == NS - 1)
    def _zero_tail():
        pltpu.sync_copy(rows_v.at[pl.ds(0, TAIL)],
                        acc_s.at[pl.ds(TAIL_BASE, TAIL)])
    plsc.subcore_barrier()

    pre_t = pre_h.at[cid]

    for blk in range(NBLK):
        pltpu.sync_copy(src_h.at[sid, pl.ds(blk * CBLK, CBLK)], src_v)
        pltpu.sync_copy(dst_h.at[sid, pl.ds(blk * CBLK, CBLK)], dst_v)
        pltpu.sync_copy(val_h.at[sid, pl.ds(blk * CBLK, CBLK)], val_v)

        def _chunk(j, _):
            pltpu.async_copy(pre_t.at[src_v.at[j]], rows_v, sem).wait()

            def _scale(g, _):
                vv = val_v[j, pl.ds(g * 16, 16)]
                for e16 in range(16):
                    lane = jnp.full((16,), e16, jnp.int32)
                    vb = vv.at[lane].get(mode="promise_in_bounds")
                    e = g * 16 + e16
                    for d in range(DH // 16):
                        sl = (e, pl.ds(d * 16, 16))
                        rows_v[sl] = rows_v[sl] * vb
                return 0
            lax.fori_loop(0, CB // 16, _scale, 0)

            pltpu.sync_copy(rows_v, acc_s.at[dst_v.at[j]], add=True)
            return 0
        lax.fori_loop(0, CBLK, _chunk, 0)

    plsc.subcore_barrier()
    pltpu.sync_copy(acc_s.at[pl.ds(base, ROWS_PER_TILE)],
                    out_h.at[cid, pl.ds(base, ROWS_PER_TILE)])

    @pl.when(sid == NS - 1)
    def _write_tail():
        pltpu.sync_copy(acc_s.at[pl.ds(TAIL_BASE, TAIL)],
                        out_h.at[cid, pl.ds(TAIL_BASE, TAIL)])


def kernel(x, edge_index_0, edge_vals_0, edge_index_1, edge_vals_1, W0, W1, b):
    wstk = jnp.stack([W0, W1])
    blk = 1000
    pre = pl.pallas_call(
        _matmul_body,
        grid=(N // blk,),
        in_specs=[
            pl.BlockSpec((blk, D), lambda i: (i, 0)),
            pl.BlockSpec((2, D, D), lambda i: (0, 0, 0)),
        ],
        out_specs=pl.BlockSpec((2, 2, blk, DH), lambda i: (0, 0, i, 0)),
        out_shape=jax.ShapeDtypeStruct((2, 2, N, DH), jnp.float32),
    )(x, wstk)
    pre_halves = pre.reshape(2, 2 * N, DH)

    src = jnp.concatenate([edge_index_0[1], edge_index_1[1] + N])
    dst = jnp.concatenate([edge_index_0[0], edge_index_1[0]])
    val = jnp.concatenate([edge_vals_0, edge_vals_1])
    pad = EPAD - 2 * E
    src = jnp.concatenate([src, jnp.zeros((pad,), jnp.int32)]).reshape(
        NS, TCH, CB)
    dst = jnp.concatenate([dst, jnp.zeros((pad,), jnp.int32)]).reshape(
        NS, TCH, CB)
    val = jnp.concatenate([val, jnp.zeros((pad,), jnp.float32)]).reshape(
        NS, TCH, CB)

    mesh = plsc.VectorSubcoreMesh(core_axis_name="c", subcore_axis_name="s")
    agg = pl.kernel(
        _sc_agg,
        out_type=jax.ShapeDtypeStruct((NC, N, DH), jnp.float32),
        mesh=mesh,
        scratch_types=[
            pltpu.VMEM((CBLK, CB), jnp.int32),
            pltpu.VMEM((CBLK, CB), jnp.int32),
            pltpu.VMEM((CBLK, CB), jnp.float32),
            pltpu.VMEM((CB, DH), jnp.float32),
            pltpu.VMEM_SHARED((N, DH), jnp.float32),
            pltpu.SemaphoreType.DMA,
        ],
        compiler_params=pltpu.CompilerParams(use_tc_tiling_on_sc=False),
    )
    acc = agg(src, dst, val, pre_halves)

    out = pl.pallas_call(
        _combine_body,
        grid=(N // blk,),
        in_specs=[
            pl.BlockSpec((2, blk, DH), lambda i: (0, i, 0)),
            pl.BlockSpec((1, D), lambda i: (0, 0)),
        ],
        out_specs=pl.BlockSpec((blk, D), lambda i: (i, 0)),
        out_shape=jax.ShapeDtypeStruct((N, D), jnp.float32),
    )(acc, b.reshape(1, D))
    return out


# speedup vs baseline: 2.5706x; 1.1670x over previous
import jax
import jax.numpy as jnp
from jax import lax
from jax.experimental import pallas as pl
from jax.experimental.pallas import tpu as pltpu
from jax.experimental.pallas import tpu_sc as plsc

N = 10000
E = 320000
D = 128
DH = D // 2
NC, NS = 2, 16
CB = 128
TCH = 320
NBLK, CBLK = 4, 80
EPAD = NS * TCH * CB
ROWS_PER_TILE = 624
TAIL_BASE = NS * ROWS_PER_TILE
TAIL = N - TAIL_BASE


def _matmul_body(x_ref, w_ref, o_ref):
    x = x_ref[...]
    p0 = jnp.dot(x, w_ref[0], preferred_element_type=jnp.float32)
    p1 = jnp.dot(x, w_ref[1], preferred_element_type=jnp.float32)
    o_ref[0, 0] = p0[:, :DH]
    o_ref[0, 1] = p1[:, :DH]
    o_ref[1, 0] = p0[:, DH:]
    o_ref[1, 1] = p1[:, DH:]


def _combine_body(a_ref, b_ref, o_ref):
    full = jnp.concatenate([a_ref[0], a_ref[1]], axis=-1)
    o_ref[...] = jnp.maximum(full + b_ref[...], 0.0)


NBUF = 4


def _sc_agg(src_h, dst_h, val_h, pre_h, out_h, src_v, dst_v, val_v,
            rows_bufs, gsems, ssems, acc_s):
    cid = lax.axis_index("c")
    sid = lax.axis_index("s")
    rows_v = rows_bufs[0]

    def _zero_rows(e, _):
        zero = jnp.zeros((16,), jnp.float32)
        for d in range(DH // 16):
            rows_v[e, pl.ds(d * 16, 16)] = zero
        return 0
    lax.fori_loop(0, CB, _zero_rows, 0)
    base = sid * ROWS_PER_TILE
    nfull = ROWS_PER_TILE // CB
    rem = ROWS_PER_TILE - nfull * CB
    for k in range(nfull):
        pltpu.sync_copy(rows_v, acc_s.at[pl.ds(base + k * CB, CB)])
    pltpu.sync_copy(rows_v.at[pl.ds(0, rem)],
                    acc_s.at[pl.ds(base + nfull * CB, rem)])

    @pl.when(sid == NS - 1)
    def _zero_tail():
        pltpu.sync_copy(rows_v.at[pl.ds(0, TAIL)],
                        acc_s.at[pl.ds(TAIL_BASE, TAIL)])
    plsc.subcore_barrier()

    pre_t = pre_h.at[cid]

    for blk in range(NBLK):
        pltpu.sync_copy(src_h.at[sid, pl.ds(blk * CBLK, CBLK)], src_v)
        pltpu.sync_copy(dst_h.at[sid, pl.ds(blk * CBLK, CBLK)], dst_v)
        pltpu.sync_copy(val_h.at[sid, pl.ds(blk * CBLK, CBLK)], val_v)

        def _scale(buf, j):
            def _grp(g, _):
                vv = val_v[j, pl.ds(g * 16, 16)]
                for e16 in range(16):
                    lane = jnp.full((16,), e16, jnp.int32)
                    vb = vv.at[lane].get(mode="promise_in_bounds")
                    e = g * 16 + e16
                    for d in range(DH // 16):
                        sl = (e, pl.ds(d * 16, 16))
                        buf[sl] = buf[sl] * vb
                return 0
            lax.fori_loop(0, CB // 16, _grp, 0)

        def _chunk(k, _):
            gds, sds = [], []
            for u in range(NBUF):
                j = k * NBUF + u
                gds.append(pltpu.async_copy(
                    pre_t.at[src_v.at[j]], rows_bufs[u], gsems[u]))
            for u in range(NBUF):
                j = k * NBUF + u
                gds[u].wait()
                _scale(rows_bufs[u], j)
                sds.append(pltpu.async_copy(
                    rows_bufs[u], acc_s.at[dst_v.at[j]], ssems[u], add=True))
            for u in range(NBUF):
                sds[u].wait()
            return 0
        lax.fori_loop(0, CBLK // NBUF, _chunk, 0)

    plsc.subcore_barrier()
    pltpu.sync_copy(acc_s.at[pl.ds(base, ROWS_PER_TILE)],
                    out_h.at[cid, pl.ds(base, ROWS_PER_TILE)])

    @pl.when(sid == NS - 1)
    def _write_tail():
        pltpu.sync_copy(acc_s.at[pl.ds(TAIL_BASE, TAIL)],
                        out_h.at[cid, pl.ds(TAIL_BASE, TAIL)])


def kernel(x, edge_index_0, edge_vals_0, edge_index_1, edge_vals_1, W0, W1, b):
    wstk = jnp.stack([W0, W1])
    blk = 1000
    pre = pl.pallas_call(
        _matmul_body,
        grid=(N // blk,),
        in_specs=[
            pl.BlockSpec((blk, D), lambda i: (i, 0)),
            pl.BlockSpec((2, D, D), lambda i: (0, 0, 0)),
        ],
        out_specs=pl.BlockSpec((2, 2, blk, DH), lambda i: (0, 0, i, 0)),
        out_shape=jax.ShapeDtypeStruct((2, 2, N, DH), jnp.float32),
    )(x, wstk)
    pre_halves = pre.reshape(2, 2 * N, DH)

    src = jnp.concatenate([edge_index_0[1], edge_index_1[1] + N])
    dst = jnp.concatenate([edge_index_0[0], edge_index_1[0]])
    val = jnp.concatenate([edge_vals_0, edge_vals_1])
    pad = EPAD - 2 * E
    src = jnp.concatenate([src, jnp.zeros((pad,), jnp.int32)]).reshape(
        NS, TCH, CB)
    dst = jnp.concatenate([dst, jnp.zeros((pad,), jnp.int32)]).reshape(
        NS, TCH, CB)
    val = jnp.concatenate([val, jnp.zeros((pad,), jnp.float32)]).reshape(
        NS, TCH, CB)

    mesh = plsc.VectorSubcoreMesh(core_axis_name="c", subcore_axis_name="s")
    agg = pl.kernel(
        _sc_agg,
        out_type=jax.ShapeDtypeStruct((NC, N, DH), jnp.float32),
        mesh=mesh,
        scratch_types=[
            pltpu.VMEM((CBLK, CB), jnp.int32),
            pltpu.VMEM((CBLK, CB), jnp.int32),
            pltpu.VMEM((CBLK, CB), jnp.float32),
            [pltpu.VMEM((CB, DH), jnp.float32) for _ in range(NBUF)],
            [pltpu.SemaphoreType.DMA for _ in range(NBUF)],
            [pltpu.SemaphoreType.DMA for _ in range(NBUF)],
            pltpu.VMEM_SHARED((N, DH), jnp.float32),
        ],
        compiler_params=pltpu.CompilerParams(use_tc_tiling_on_sc=False),
    )
    acc = agg(src, dst, val, pre_halves)

    out = pl.pallas_call(
        _combine_body,
        grid=(N // blk,),
        in_specs=[
            pl.BlockSpec((2, blk, DH), lambda i: (0, i, 0)),
            pl.BlockSpec((1, D), lambda i: (0, 0)),
        ],
        out_specs=pl.BlockSpec((blk, D), lambda i: (i, 0)),
        out_shape=jax.ShapeDtypeStruct((N, D), jnp.float32),
    )(acc, b.reshape(1, D))
    return out


# speedup vs baseline: 2.7966x; 1.0879x over previous
import jax
import jax.numpy as jnp
from jax import lax
from jax.experimental import pallas as pl
from jax.experimental.pallas import tpu as pltpu
from jax.experimental.pallas import tpu_sc as plsc

N = 10000
E = 320000
D = 128
DH = D // 2
NC, NS = 2, 16
CB = 128
TCH = 320
NBLK, CBLK = 4, 80
EPAD = NS * TCH * CB
ROWS_PER_TILE = 624
TAIL_BASE = NS * ROWS_PER_TILE
TAIL = N - TAIL_BASE


def _matmul_body(x_ref, w_ref, o_ref):
    x = x_ref[...]
    p0 = jnp.dot(x, w_ref[0], preferred_element_type=jnp.float32)
    p1 = jnp.dot(x, w_ref[1], preferred_element_type=jnp.float32)
    o_ref[0, 0] = p0[:, :DH]
    o_ref[0, 1] = p1[:, :DH]
    o_ref[1, 0] = p0[:, DH:]
    o_ref[1, 1] = p1[:, DH:]


def _combine_body(a_ref, b_ref, o_ref):
    full = jnp.concatenate([a_ref[0], a_ref[1]], axis=-1)
    o_ref[...] = jnp.maximum(full + b_ref[...], 0.0)


NBUF = 4


def _sc_agg(src_h, dst_h, val_h, pre_h, out_h, src_v, dst_v, val_v,
            rows_bufs, gsems, ssems, acc_s):
    cid = lax.axis_index("c")
    sid = lax.axis_index("s")
    rows_v = rows_bufs[0]

    def _zero_rows(e, _):
        zero = jnp.zeros((16,), jnp.float32)
        for d in range(DH // 16):
            rows_v[e, pl.ds(d * 16, 16)] = zero
        return 0
    lax.fori_loop(0, CB, _zero_rows, 0)
    base = sid * ROWS_PER_TILE
    nfull = ROWS_PER_TILE // CB
    rem = ROWS_PER_TILE - nfull * CB
    for k in range(nfull):
        pltpu.sync_copy(rows_v, acc_s.at[pl.ds(base + k * CB, CB)])
    pltpu.sync_copy(rows_v.at[pl.ds(0, rem)],
                    acc_s.at[pl.ds(base + nfull * CB, rem)])

    @pl.when(sid == NS - 1)
    def _zero_tail():
        pltpu.sync_copy(rows_v.at[pl.ds(0, TAIL)],
                        acc_s.at[pl.ds(TAIL_BASE, TAIL)])
    plsc.subcore_barrier()

    pre_t = pre_h.at[cid]

    for blk in range(NBLK):
        pltpu.sync_copy(src_h.at[sid, pl.ds(blk * CBLK, CBLK)], src_v)
        pltpu.sync_copy(dst_h.at[sid, pl.ds(blk * CBLK, CBLK)], dst_v)
        pltpu.sync_copy(val_h.at[sid, pl.ds(blk * CBLK, CBLK)], val_v)

        def _scale(buf, j):
            def _grp(g, _):
                vv = val_v[j, pl.ds(g * 16, 16)]
                for e16 in range(16):
                    lane = jnp.full((16,), e16, jnp.int32)
                    vb = vv.at[lane].get(mode="promise_in_bounds")
                    e = g * 16 + e16
                    for d in range(DH // 16):
                        sl = (e, pl.ds(d * 16, 16))
                        buf[sl] = buf[sl] * vb
                return 0
            lax.fori_loop(0, CB // 16, _grp, 0)

        def _gfire(j, u):
            pltpu.async_copy(pre_t.at[src_v.at[j]], rows_bufs[u], gsems[u])

        def _drain(j, u, sem):
            pltpu.make_async_copy(pre_t.at[src_v.at[j]], rows_bufs[u],
                                  sem).wait()

        def _sfire(j, u):
            pltpu.async_copy(rows_bufs[u], acc_s.at[dst_v.at[j]], ssems[u],
                             add=True)

        niter = CBLK // NBUF
        for u in range(NBUF):
            _gfire(u, u)

        def _steady(k, _):
            for u in range(NBUF):
                j = k * NBUF + u
                _drain(j, u, gsems[u])
                _scale(rows_bufs[u], j)
                _sfire(j, u)
            for u in range(NBUF):
                jn = (k + 1) * NBUF + u
                _drain(jn, u, ssems[u])
                _gfire(jn, u)
            return 0
        lax.fori_loop(0, niter - 1, _steady, 0)

        for u in range(NBUF):
            j = (niter - 1) * NBUF + u
            _drain(j, u, gsems[u])
            _scale(rows_bufs[u], j)
            _sfire(j, u)
        for u in range(NBUF):
            _drain((niter - 1) * NBUF + u, u, ssems[u])

    plsc.subcore_barrier()
    pltpu.sync_copy(acc_s.at[pl.ds(base, ROWS_PER_TILE)],
                    out_h.at[cid, pl.ds(base, ROWS_PER_TILE)])

    @pl.when(sid == NS - 1)
    def _write_tail():
        pltpu.sync_copy(acc_s.at[pl.ds(TAIL_BASE, TAIL)],
                        out_h.at[cid, pl.ds(TAIL_BASE, TAIL)])


def kernel(x, edge_index_0, edge_vals_0, edge_index_1, edge_vals_1, W0, W1, b):
    wstk = jnp.stack([W0, W1])
    blk = 1000
    pre = pl.pallas_call(
        _matmul_body,
        grid=(N // blk,),
        in_specs=[
            pl.BlockSpec((blk, D), lambda i: (i, 0)),
            pl.BlockSpec((2, D, D), lambda i: (0, 0, 0)),
        ],
        out_specs=pl.BlockSpec((2, 2, blk, DH), lambda i: (0, 0, i, 0)),
        out_shape=jax.ShapeDtypeStruct((2, 2, N, DH), jnp.float32),
    )(x, wstk)
    pre_halves = pre.reshape(2, 2 * N, DH)

    src = jnp.concatenate([edge_index_0[1], edge_index_1[1] + N])
    dst = jnp.concatenate([edge_index_0[0], edge_index_1[0]])
    val = jnp.concatenate([edge_vals_0, edge_vals_1])
    pad = EPAD - 2 * E
    src = jnp.concatenate([src, jnp.zeros((pad,), jnp.int32)]).reshape(
        NS, TCH, CB)
    dst = jnp.concatenate([dst, jnp.zeros((pad,), jnp.int32)]).reshape(
        NS, TCH, CB)
    val = jnp.concatenate([val, jnp.zeros((pad,), jnp.float32)]).reshape(
        NS, TCH, CB)

    mesh = plsc.VectorSubcoreMesh(core_axis_name="c", subcore_axis_name="s")
    agg = pl.kernel(
        _sc_agg,
        out_type=jax.ShapeDtypeStruct((NC, N, DH), jnp.float32),
        mesh=mesh,
        scratch_types=[
            pltpu.VMEM((CBLK, CB), jnp.int32),
            pltpu.VMEM((CBLK, CB), jnp.int32),
            pltpu.VMEM((CBLK, CB), jnp.float32),
            [pltpu.VMEM((CB, DH), jnp.float32) for _ in range(NBUF)],
            [pltpu.SemaphoreType.DMA for _ in range(NBUF)],
            [pltpu.SemaphoreType.DMA for _ in range(NBUF)],
            pltpu.VMEM_SHARED((N, DH), jnp.float32),
        ],
        compiler_params=pltpu.CompilerParams(use_tc_tiling_on_sc=False),
    )
    acc = agg(src, dst, val, pre_halves)

    out = pl.pallas_call(
        _combine_body,
        grid=(N // blk,),
        in_specs=[
            pl.BlockSpec((2, blk, DH), lambda i: (0, i, 0)),
            pl.BlockSpec((1, D), lambda i: (0, 0)),
        ],
        out_specs=pl.BlockSpec((blk, D), lambda i: (i, 0)),
        out_shape=jax.ShapeDtypeStruct((N, D), jnp.float32),
    )(acc, b.reshape(1, D))
    return out


# speedup vs baseline: 3.5718x; 1.2772x over previous
import jax
import jax.numpy as jnp
from jax import lax
from jax.experimental import pallas as pl
from jax.experimental.pallas import tpu as pltpu
from jax.experimental.pallas import tpu_sc as plsc

N = 10000
E = 320000
D = 128
DH = D // 2
NC, NS = 2, 16
CB = 128
TCH = 320
NBLK, CBLK = 4, 80
EPAD = NS * TCH * CB
ROWS_PER_TILE = 624
TAIL_BASE = NS * ROWS_PER_TILE
TAIL = N - TAIL_BASE


def _matmul_body(x_ref, w_ref, o_ref):
    x = x_ref[...]
    p0 = jnp.dot(x, w_ref[0], preferred_element_type=jnp.float32)
    p1 = jnp.dot(x, w_ref[1], preferred_element_type=jnp.float32)
    o_ref[0, 0] = p0[:, :DH]
    o_ref[0, 1] = p1[:, :DH]
    o_ref[1, 0] = p0[:, DH:]
    o_ref[1, 1] = p1[:, DH:]


def _combine_body(a_ref, b_ref, o_ref):
    full = jnp.concatenate([a_ref[0], a_ref[1]], axis=-1)
    o_ref[...] = jnp.maximum(full + b_ref[...], 0.0)


NBUF = 4


def _sc_agg(src_h, dst_h, val_h, pre_h, out_h, src_v, dst_v, val_v,
            rows_bufs, zbuf, gsems, ssems, acc_s):
    cid = lax.axis_index("c")
    sid = lax.axis_index("s")
    rows_v = zbuf

    def _zero_rows(e, _):
        zero = jnp.zeros((16,), jnp.float32)
        for d in range(DH // 16):
            rows_v[e, pl.ds(d * 16, 16)] = zero
        return 0
    lax.fori_loop(0, CB, _zero_rows, 0)
    base = sid * ROWS_PER_TILE
    nfull = ROWS_PER_TILE // CB
    rem = ROWS_PER_TILE - nfull * CB
    for k in range(nfull):
        pltpu.sync_copy(rows_v, acc_s.at[pl.ds(base + k * CB, CB)])
    pltpu.sync_copy(rows_v.at[pl.ds(0, rem)],
                    acc_s.at[pl.ds(base + nfull * CB, rem)])

    @pl.when(sid == NS - 1)
    def _zero_tail():
        pltpu.sync_copy(rows_v.at[pl.ds(0, TAIL)],
                        acc_s.at[pl.ds(TAIL_BASE, TAIL)])
    plsc.subcore_barrier()

    pre_t = pre_h.at[cid]

    for blk in range(NBLK):
        pltpu.sync_copy(src_h.at[sid, pl.ds(blk * CBLK, CBLK)], src_v)
        pltpu.sync_copy(dst_h.at[sid, pl.ds(blk * CBLK, CBLK)], dst_v)
        pltpu.sync_copy(val_h.at[sid, pl.ds(blk * CBLK, CBLK)], val_v)

        def _scale(buf, j):
            def _grp(g, _):
                vv = val_v[j, pl.ds(g * 16, 16)]
                for e16 in range(16):
                    lane = jnp.full((16,), e16, jnp.int32)
                    vb = vv.at[lane].get(mode="promise_in_bounds")
                    e = g * 16 + e16
                    for d in range(DH // 16):
                        sl = (e, pl.ds(d * 16, 16))
                        buf[sl] = buf[sl] * vb
                return 0
            lax.fori_loop(0, CB // 16, _grp, 0)

        def _gfire(j, u):
            pltpu.async_copy(pre_t.at[src_v.at[j]], rows_bufs[u], gsems[u])

        def _drain(u, sem):
            pltpu.make_async_copy(pre_t.at[src_v.at[0]], rows_bufs[u],
                                  sem).wait()

        def _sfire(j, u):
            pltpu.async_copy(rows_bufs[u], acc_s.at[dst_v.at[j]], ssems[u],
                             add=True)

        H = NBUF // 2
        niter = CBLK // NBUF
        for u in range(H, NBUF):
            pltpu.async_copy(zbuf, acc_s.at[dst_v.at[0]], ssems[u], add=True)
        for u in range(H):
            _gfire(u, u)

        def _steady(k, _):
            for u in range(NBUF):
                j = k * NBUF + u
                uf = (u + H) % NBUF
                if u < H:
                    _drain(uf, ssems[uf])
                    _gfire(j + H, uf)
                else:
                    @pl.when(k < niter - 1)
                    def _pf():
                        _drain(uf, ssems[uf])
                        _gfire(j + H, uf)
                _drain(u, gsems[u])
                _scale(rows_bufs[u], j)
                _sfire(j, u)
            return 0
        lax.fori_loop(0, niter, _steady, 0)

        for u in range(H, NBUF):
            _drain(u, ssems[u])

    plsc.subcore_barrier()
    pltpu.sync_copy(acc_s.at[pl.ds(base, ROWS_PER_TILE)],
                    out_h.at[cid, pl.ds(base, ROWS_PER_TILE)])

    @pl.when(sid == NS - 1)
    def _write_tail():
        pltpu.sync_copy(acc_s.at[pl.ds(TAIL_BASE, TAIL)],
                        out_h.at[cid, pl.ds(TAIL_BASE, TAIL)])


def kernel(x, edge_index_0, edge_vals_0, edge_index_1, edge_vals_1, W0, W1, b):
    wstk = jnp.stack([W0, W1])
    blk = 1000
    pre = pl.pallas_call(
        _matmul_body,
        grid=(N // blk,),
        in_specs=[
            pl.BlockSpec((blk, D), lambda i: (i, 0)),
            pl.BlockSpec((2, D, D), lambda i: (0, 0, 0)),
        ],
        out_specs=pl.BlockSpec((2, 2, blk, DH), lambda i: (0, 0, i, 0)),
        out_shape=jax.ShapeDtypeStruct((2, 2, N, DH), jnp.float32),
    )(x, wstk)
    pre_halves = pre.reshape(2, 2 * N, DH)

    src = jnp.concatenate([edge_index_0[1], edge_index_1[1] + N])
    dst = jnp.concatenate([edge_index_0[0], edge_index_1[0]])
    val = jnp.concatenate([edge_vals_0, edge_vals_1])
    pad = EPAD - 2 * E
    src = jnp.concatenate([src, jnp.zeros((pad,), jnp.int32)]).reshape(
        NS, TCH, CB)
    dst = jnp.concatenate([dst, jnp.zeros((pad,), jnp.int32)]).reshape(
        NS, TCH, CB)
    val = jnp.concatenate([val, jnp.zeros((pad,), jnp.float32)]).reshape(
        NS, TCH, CB)

    mesh = plsc.VectorSubcoreMesh(core_axis_name="c", subcore_axis_name="s")
    agg = pl.kernel(
        _sc_agg,
        out_type=jax.ShapeDtypeStruct((NC, N, DH), jnp.float32),
        mesh=mesh,
        scratch_types=[
            pltpu.VMEM((CBLK, CB), jnp.int32),
            pltpu.VMEM((CBLK, CB), jnp.int32),
            pltpu.VMEM((CBLK, CB), jnp.float32),
            [pltpu.VMEM((CB, DH), jnp.float32) for _ in range(NBUF)],
            pltpu.VMEM((CB, DH), jnp.float32),
            [pltpu.SemaphoreType.DMA for _ in range(NBUF)],
            [pltpu.SemaphoreType.DMA for _ in range(NBUF)],
            pltpu.VMEM_SHARED((N, DH), jnp.float32),
        ],
        compiler_params=pltpu.CompilerParams(use_tc_tiling_on_sc=False),
    )
    acc = agg(src, dst, val, pre_halves)

    out = pl.pallas_call(
        _combine_body,
        grid=(N // blk,),
        in_specs=[
            pl.BlockSpec((2, blk, DH), lambda i: (0, i, 0)),
            pl.BlockSpec((1, D), lambda i: (0, 0)),
        ],
        out_specs=pl.BlockSpec((blk, D), lambda i: (i, 0)),
        out_shape=jax.ShapeDtypeStruct((N, D), jnp.float32),
    )(acc, b.reshape(1, D))
    return out


# speedup vs baseline: 5.0411x; 1.4114x over previous
import jax
import jax.numpy as jnp
from jax import lax
from jax.experimental import pallas as pl
from jax.experimental.pallas import tpu as pltpu
from jax.experimental.pallas import tpu_sc as plsc

N = 10000
E = 320000
D = 128
DH = D // 2
NC, NS = 2, 16
CB = 128
TCH = 320
NBLK, CBLK = 4, 80
EPAD = NS * TCH * CB
ROWS_PER_TILE = 624
TAIL_BASE = NS * ROWS_PER_TILE
TAIL = N - TAIL_BASE


def _matmul_body(x_ref, w_ref, o_ref):
    x = x_ref[...]
    p0 = jnp.dot(x, w_ref[0], preferred_element_type=jnp.float32)
    p1 = jnp.dot(x, w_ref[1], preferred_element_type=jnp.float32)
    o_ref[0, 0] = p0[:, :DH]
    o_ref[0, 1] = p1[:, :DH]
    o_ref[1, 0] = p0[:, DH:]
    o_ref[1, 1] = p1[:, DH:]


def _combine_body(a_ref, b_ref, o_ref):
    full = jnp.concatenate([a_ref[0], a_ref[1]], axis=-1)
    o_ref[...] = jnp.maximum(full + b_ref[...], 0.0)


NBUF = 4


def _sc_agg(src_h, dst_h, val_h, pre_h, out_h, src_v, dst_v, val_v,
            rows_bufs, zbuf, gsems, ssems, acc_s):
    cid = lax.axis_index("c")
    sid = lax.axis_index("s")
    rows_v = zbuf

    def _zero_rows(e, _):
        zero = jnp.zeros((16,), jnp.float32)
        for d in range(DH // 16):
            rows_v[e, pl.ds(d * 16, 16)] = zero
        return 0
    lax.fori_loop(0, CB, _zero_rows, 0)
    base = sid * ROWS_PER_TILE
    nfull = ROWS_PER_TILE // CB
    rem = ROWS_PER_TILE - nfull * CB
    for k in range(nfull):
        pltpu.sync_copy(rows_v, acc_s.at[pl.ds(base + k * CB, CB)])
    pltpu.sync_copy(rows_v.at[pl.ds(0, rem)],
                    acc_s.at[pl.ds(base + nfull * CB, rem)])

    @pl.when(sid == NS - 1)
    def _zero_tail():
        pltpu.sync_copy(rows_v.at[pl.ds(0, TAIL)],
                        acc_s.at[pl.ds(TAIL_BASE, TAIL)])
    plsc.subcore_barrier()

    pre_t = pre_h.at[cid]

    for blk in range(NBLK):
        pltpu.sync_copy(src_h.at[sid, pl.ds(blk * CBLK, CBLK)], src_v)
        pltpu.sync_copy(dst_h.at[sid, pl.ds(blk * CBLK, CBLK)], dst_v)
        pltpu.sync_copy(val_h.at[sid, pl.ds(blk * CBLK, CBLK)], val_v)

        def _scale(buf, j):
            @plsc.parallel_loop(0, CB // 16, 1, unroll=2)
            def _grp(g):
                vv = val_v[j, pl.ds(g * 16, 16)]
                for e16 in range(16):
                    lane = jnp.full((16,), e16, jnp.int32)
                    vb = vv.at[lane].get(mode="promise_in_bounds")
                    e = g * 16 + e16
                    for d in range(DH // 16):
                        sl = (e, pl.ds(d * 16, 16))
                        buf[sl] = buf[sl] * vb

        def _gfire(j, u):
            pltpu.async_copy(pre_t.at[src_v.at[j]], rows_bufs[u], gsems[u])

        def _drain(u, sem):
            pltpu.make_async_copy(pre_t.at[src_v.at[0]], rows_bufs[u],
                                  sem).wait()

        def _sfire(j, u):
            pltpu.async_copy(rows_bufs[u], acc_s.at[dst_v.at[j]], ssems[u],
                             add=True)

        H = NBUF // 2
        niter = CBLK // NBUF
        for u in range(H, NBUF):
            pltpu.async_copy(zbuf, acc_s.at[dst_v.at[0]], ssems[u], add=True)
        for u in range(H):
            _gfire(u, u)

        def _steady(k, _):
            for u in range(NBUF):
                j = k * NBUF + u
                uf = (u + H) % NBUF
                if u < H:
                    _drain(uf, ssems[uf])
                    _gfire(j + H, uf)
                else:
                    @pl.when(k < niter - 1)
                    def _pf():
                        _drain(uf, ssems[uf])
                        _gfire(j + H, uf)
                _drain(u, gsems[u])
                _scale(rows_bufs[u], j)
                _sfire(j, u)
            return 0
        lax.fori_loop(0, niter, _steady, 0)

        for u in range(H, NBUF):
            _drain(u, ssems[u])

    plsc.subcore_barrier()
    pltpu.sync_copy(acc_s.at[pl.ds(base, ROWS_PER_TILE)],
                    out_h.at[cid, pl.ds(base, ROWS_PER_TILE)])

    @pl.when(sid == NS - 1)
    def _write_tail():
        pltpu.sync_copy(acc_s.at[pl.ds(TAIL_BASE, TAIL)],
                        out_h.at[cid, pl.ds(TAIL_BASE, TAIL)])


def kernel(x, edge_index_0, edge_vals_0, edge_index_1, edge_vals_1, W0, W1, b):
    wstk = jnp.stack([W0, W1])
    blk = 1000
    pre = pl.pallas_call(
        _matmul_body,
        grid=(N // blk,),
        in_specs=[
            pl.BlockSpec((blk, D), lambda i: (i, 0)),
            pl.BlockSpec((2, D, D), lambda i: (0, 0, 0)),
        ],
        out_specs=pl.BlockSpec((2, 2, blk, DH), lambda i: (0, 0, i, 0)),
        out_shape=jax.ShapeDtypeStruct((2, 2, N, DH), jnp.float32),
    )(x, wstk)
    pre_halves = pre.reshape(2, 2 * N, DH)

    src = jnp.concatenate([edge_index_0[1], edge_index_1[1] + N])
    dst = jnp.concatenate([edge_index_0[0], edge_index_1[0]])
    val = jnp.concatenate([edge_vals_0, edge_vals_1])
    pad = EPAD - 2 * E
    src = jnp.concatenate([src, jnp.zeros((pad,), jnp.int32)]).reshape(
        NS, TCH, CB)
    dst = jnp.concatenate([dst, jnp.zeros((pad,), jnp.int32)]).reshape(
        NS, TCH, CB)
    val = jnp.concatenate([val, jnp.zeros((pad,), jnp.float32)]).reshape(
        NS, TCH, CB)

    mesh = plsc.VectorSubcoreMesh(core_axis_name="c", subcore_axis_name="s")
    agg = pl.kernel(
        _sc_agg,
        out_type=jax.ShapeDtypeStruct((NC, N, DH), jnp.float32),
        mesh=mesh,
        scratch_types=[
            pltpu.VMEM((CBLK, CB), jnp.int32),
            pltpu.VMEM((CBLK, CB), jnp.int32),
            pltpu.VMEM((CBLK, CB), jnp.float32),
            [pltpu.VMEM((CB, DH), jnp.float32) for _ in range(NBUF)],
            pltpu.VMEM((CB, DH), jnp.float32),
            [pltpu.SemaphoreType.DMA for _ in range(NBUF)],
            [pltpu.SemaphoreType.DMA for _ in range(NBUF)],
            pltpu.VMEM_SHARED((N, DH), jnp.float32),
        ],
        compiler_params=pltpu.CompilerParams(use_tc_tiling_on_sc=False),
    )
    acc = agg(src, dst, val, pre_halves)

    out = pl.pallas_call(
        _combine_body,
        grid=(N // blk,),
        in_specs=[
            pl.BlockSpec((2, blk, DH), lambda i: (0, i, 0)),
            pl.BlockSpec((1, D), lambda i: (0, 0)),
        ],
        out_specs=pl.BlockSpec((blk, D), lambda i: (i, 0)),
        out_shape=jax.ShapeDtypeStruct((N, D), jnp.float32),
    )(acc, b.reshape(1, D))
    return out


# speedup vs baseline: 5.0444x; 1.0007x over previous
import jax
import jax.numpy as jnp
from jax import lax
from jax.experimental import pallas as pl
from jax.experimental.pallas import tpu as pltpu
from jax.experimental.pallas import tpu_sc as plsc

N = 10000
E = 320000
D = 128
DH = D // 2
NC, NS = 2, 16
CB = 128
TCH = 320
NBLK, CBLK = 4, 80
EPAD = NS * TCH * CB
ROWS_PER_TILE = 624
TAIL_BASE = NS * ROWS_PER_TILE
TAIL = N - TAIL_BASE


def _matmul_body(x_ref, w_ref, o_ref):
    x = x_ref[...]
    p0 = jnp.dot(x, w_ref[0], preferred_element_type=jnp.float32)
    p1 = jnp.dot(x, w_ref[1], preferred_element_type=jnp.float32)
    o_ref[0, 0] = p0[:, :DH]
    o_ref[0, 1] = p1[:, :DH]
    o_ref[1, 0] = p0[:, DH:]
    o_ref[1, 1] = p1[:, DH:]


def _combine_body(a_ref, b_ref, o_ref):
    full = jnp.concatenate([a_ref[0], a_ref[1]], axis=-1)
    o_ref[...] = jnp.maximum(full + b_ref[...], 0.0)


NBUF = 4


def _sc_agg(src_h, dst_h, val_h, pre_h, out_h, src_v, dst_v, val_v,
            rows_bufs, zbuf, gsems, ssems, acc_s):
    cid = lax.axis_index("c")
    sid = lax.axis_index("s")
    rows_v = zbuf

    def _zero_rows(e, _):
        zero = jnp.zeros((16,), jnp.float32)
        for d in range(DH // 16):
            rows_v[e, pl.ds(d * 16, 16)] = zero
        return 0
    lax.fori_loop(0, CB, _zero_rows, 0)
    base = sid * ROWS_PER_TILE
    nfull = ROWS_PER_TILE // CB
    rem = ROWS_PER_TILE - nfull * CB
    for k in range(nfull):
        pltpu.sync_copy(rows_v, acc_s.at[pl.ds(base + k * CB, CB)])
    pltpu.sync_copy(rows_v.at[pl.ds(0, rem)],
                    acc_s.at[pl.ds(base + nfull * CB, rem)])

    @pl.when(sid == NS - 1)
    def _zero_tail():
        pltpu.sync_copy(rows_v.at[pl.ds(0, TAIL)],
                        acc_s.at[pl.ds(TAIL_BASE, TAIL)])
    plsc.subcore_barrier()

    pre_t = pre_h.at[cid]

    for blk in range(NBLK):
        pltpu.sync_copy(src_h.at[sid, pl.ds(blk * CBLK, CBLK)], src_v)
        pltpu.sync_copy(dst_h.at[sid, pl.ds(blk * CBLK, CBLK)], dst_v)
        pltpu.sync_copy(val_h.at[sid, pl.ds(blk * CBLK, CBLK)], val_v)

        def _scale(buf, j):
            def _grp(g2, _):
                for gh in range(2):
                    g = g2 * 2 + gh
                    vv = val_v[j, pl.ds(g * 16, 16)]
                    for e16 in range(16):
                        lane = jnp.full((16,), e16, jnp.int32)
                        vb = vv.at[lane].get(mode="promise_in_bounds")
                        e = g * 16 + e16
                        for d in range(DH // 16):
                            sl = (e, pl.ds(d * 16, 16))
                            buf[sl] = buf[sl] * vb
                return 0
            lax.fori_loop(0, CB // 32, _grp, 0)

        def _gfire(j, u):
            pltpu.async_copy(pre_t.at[src_v.at[j]], rows_bufs[u], gsems[u])

        def _drain(u, sem):
            pltpu.make_async_copy(pre_t.at[src_v.at[0]], rows_bufs[u],
                                  sem).wait()

        def _sfire(j, u):
            pltpu.async_copy(rows_bufs[u], acc_s.at[dst_v.at[j]], ssems[u],
                             add=True)

        H = NBUF // 2
        niter = CBLK // NBUF
        for u in range(H, NBUF):
            pltpu.async_copy(zbuf, acc_s.at[dst_v.at[0]], ssems[u], add=True)
        for u in range(H):
            _gfire(u, u)

        def _steady(k, _):
            for u in range(NBUF):
                j = k * NBUF + u
                uf = (u + H) % NBUF
                if u < H:
                    _drain(uf, ssems[uf])
                    _gfire(j + H, uf)
                else:
                    @pl.when(k < niter - 1)
                    def _pf():
                        _drain(uf, ssems[uf])
                        _gfire(j + H, uf)
                _drain(u, gsems[u])
                _scale(rows_bufs[u], j)
                _sfire(j, u)
            return 0
        lax.fori_loop(0, niter, _steady, 0)

        for u in range(H, NBUF):
            _drain(u, ssems[u])

    plsc.subcore_barrier()
    pltpu.sync_copy(acc_s.at[pl.ds(base, ROWS_PER_TILE)],
                    out_h.at[cid, pl.ds(base, ROWS_PER_TILE)])

    @pl.when(sid == NS - 1)
    def _write_tail():
        pltpu.sync_copy(acc_s.at[pl.ds(TAIL_BASE, TAIL)],
                        out_h.at[cid, pl.ds(TAIL_BASE, TAIL)])


def kernel(x, edge_index_0, edge_vals_0, edge_index_1, edge_vals_1, W0, W1, b):
    wstk = jnp.stack([W0, W1])
    blk = 1000
    pre = pl.pallas_call(
        _matmul_body,
        grid=(N // blk,),
        in_specs=[
            pl.BlockSpec((blk, D), lambda i: (i, 0)),
            pl.BlockSpec((2, D, D), lambda i: (0, 0, 0)),
        ],
        out_specs=pl.BlockSpec((2, 2, blk, DH), lambda i: (0, 0, i, 0)),
        out_shape=jax.ShapeDtypeStruct((2, 2, N, DH), jnp.float32),
    )(x, wstk)
    pre_halves = pre.reshape(2, 2 * N, DH)

    src = jnp.concatenate([edge_index_0[1], edge_index_1[1] + N])
    dst = jnp.concatenate([edge_index_0[0], edge_index_1[0]])
    val = jnp.concatenate([edge_vals_0, edge_vals_1])
    pad = EPAD - 2 * E
    src = jnp.concatenate([src, jnp.zeros((pad,), jnp.int32)]).reshape(
        NS, TCH, CB)
    dst = jnp.concatenate([dst, jnp.zeros((pad,), jnp.int32)]).reshape(
        NS, TCH, CB)
    val = jnp.concatenate([val, jnp.zeros((pad,), jnp.float32)]).reshape(
        NS, TCH, CB)

    mesh = plsc.VectorSubcoreMesh(core_axis_name="c", subcore_axis_name="s")
    agg = pl.kernel(
        _sc_agg,
        out_type=jax.ShapeDtypeStruct((NC, N, DH), jnp.float32),
        mesh=mesh,
        scratch_types=[
            pltpu.VMEM((CBLK, CB), jnp.int32),
            pltpu.VMEM((CBLK, CB), jnp.int32),
            pltpu.VMEM((CBLK, CB), jnp.float32),
            [pltpu.VMEM((CB, DH), jnp.float32) for _ in range(NBUF)],
            pltpu.VMEM((CB, DH), jnp.float32),
            [pltpu.SemaphoreType.DMA for _ in range(NBUF)],
            [pltpu.SemaphoreType.DMA for _ in range(NBUF)],
            pltpu.VMEM_SHARED((N, DH), jnp.float32),
        ],
        compiler_params=pltpu.CompilerParams(use_tc_tiling_on_sc=False),
    )
    acc = agg(src, dst, val, pre_halves)

    out = pl.pallas_call(
        _combine_body,
        grid=(N // blk,),
        in_specs=[
            pl.BlockSpec((2, blk, DH), lambda i: (0, i, 0)),
            pl.BlockSpec((1, D), lambda i: (0, 0)),
        ],
        out_specs=pl.BlockSpec((blk, D), lambda i: (i, 0)),
        out_shape=jax.ShapeDtypeStruct((N, D), jnp.float32),
    )(acc, b.reshape(1, D))
    return out
